# Initial kernel scaffold; baseline (speedup 1.0000x reference)
import jax
import jax.numpy as jnp
from jax.experimental import pallas as pl


def kernel(xs, params):
    raise NotImplementedError("write your pallas kernel here")


# speedup vs baseline: 12.0133x; 12.0133x over previous
import functools

import jax
import jax.numpy as jnp
from jax.experimental import pallas as pl
from jax.experimental.pallas import tpu as pltpu

D = 768
F = 12288
K = 64
TB = 64
N_BISECT = 24


def _sae_body(x_ref, wenc_ref, wdect_ref, benc_ref, bdec_ref, out_ref, post_ref):
    x = x_ref[...] - bdec_ref[...]
    pre = jax.lax.dot_general(
        x, wenc_ref[...], (((1,), (1,)), ((), ())),
        preferred_element_type=jnp.float32)
    post = jnp.maximum(pre + benc_ref[...], 0.0)
    post_ref[...] = post

    hi = jnp.max(post, axis=1, keepdims=True)
    lo = jnp.zeros_like(hi)

    def body(_, carry):
        lo, hi = carry
        mid = (lo + hi) * 0.5
        cnt = jnp.sum((post_ref[...] > mid).astype(jnp.float32), axis=1,
                      keepdims=True)
        ge = cnt >= K
        return jnp.where(ge, mid, lo), jnp.where(ge, hi, mid)

    lo, hi = jax.lax.fori_loop(0, N_BISECT, body, (lo, hi))

    masked = jnp.where(post_ref[...] > lo, post_ref[...], 0.0)
    out = jax.lax.dot_general(
        masked.astype(jnp.bfloat16), wdect_ref[...], (((1,), (0,)), ((), ())),
        preferred_element_type=jnp.float32)
    out_ref[...] = out + bdec_ref[...]


@jax.jit
def _sae_forward(x, w_enc, w_dec_t, b_enc, b_dec):
    s = x.shape[0]
    grid = (s // TB,)
    return pl.pallas_call(
        _sae_body,
        grid=grid,
        in_specs=[
            pl.BlockSpec((TB, D), lambda i: (i, 0)),
            pl.BlockSpec((F, D), lambda i: (0, 0)),
            pl.BlockSpec((F, D), lambda i: (0, 0)),
            pl.BlockSpec((1, F), lambda i: (0, 0)),
            pl.BlockSpec((1, D), lambda i: (0, 0)),
        ],
        out_specs=pl.BlockSpec((TB, D), lambda i: (i, 0)),
        out_shape=jax.ShapeDtypeStruct((s, D), jnp.float32),
        scratch_shapes=[pltpu.VMEM((TB, F), jnp.float32)],
        compiler_params=pltpu.CompilerParams(
            vmem_limit_bytes=66060288,
        ),
    )(x, w_enc, w_dec_t, b_enc, b_dec)


_NAMES = ["attn_0", "mlp_0", "attn_1", "mlp_1"]


def kernel(xs, params):
    names = [n for n in _NAMES if n in xs] or list(xs.keys())
    outs = []
    for name in names:
        p = params[name]
        x = xs[name]
        b, s, d = x.shape
        out = _sae_forward(
            x.reshape(b * s, d),
            p["W_enc"],
            p["W_dec"].T.astype(jnp.bfloat16),
            p["b_enc"].reshape(1, F),
            p["b_dec"].reshape(1, D),
        )
        outs.append(out.reshape(b, s, d))
    return jnp.stack(outs, axis=0)
